# baseline (speedup 1.0000x reference)
import functools

import jax
import jax.numpy as jnp
from jax import lax
from jax.experimental import pallas as pl
from jax.experimental.pallas import tpu as pltpu
from jax.experimental.pallas import tpu_sc as plsc

ALPHA = 0.2
BPB = 8
WIN = 72
NTILES = 32
CHUNK = 16


def _attn_kernel(offsets_ref, cxt_ref, mask_ref, feat_ref, w_ref, a2_ref,
                 out_ref, wa_ref):
    g = pl.program_id(0)
    nn = cxt_ref.shape[2]
    nrows = feat_ref.shape[0]

    @pl.when(g == 0)
    def _():
        wa_ref[...] = jnp.dot(w_ref[:], a2_ref[:].T,
                              preferred_element_type=jnp.float32)

    wa = wa_ref[...]
    feat_prog = feat_ref[pl.ds(g * BPB * nn, BPB * nn), :]
    st_t = jax.lax.dot_general(wa, feat_prog, (((0,), (1,)), ((), ())),
                               preferred_element_type=jnp.float32)

    iota_sub = jax.lax.broadcasted_iota(jnp.int32, (1, WIN, 1), 1)
    cols = []
    for k in range(BPB):
        b = g * BPB + k
        off = offsets_ref[b]
        base = jnp.minimum((off // 8) * 8, nrows - WIN)
        r = off - base

        win = feat_ref[pl.ds(base, WIN), :]
        tw = jnp.dot(win, wa[:, 1:2], preferred_element_type=jnp.float32)

        cxt = cxt_ref[k] + r
        m = mask_ref[k]
        ohm = jnp.where(cxt[:, None, :] == iota_sub, m[:, None, :], 0.0)
        tg = jnp.sum(ohm * tw[None, :, :], axis=1)

        s_row = st_t[0:1, k * nn:(k + 1) * nn]
        t_row = st_t[1:2, k * nn:(k + 1) * nn]
        e = jnp.concatenate([s_row + t_row, s_row + tg], axis=0)
        e = jnp.where(e >= 0, e, ALPHA * e)
        e = e - jnp.max(e, axis=0, keepdims=True)
        ex = jnp.exp(e)
        attn = ex / jnp.sum(ex, axis=0, keepdims=True)
        wts = jnp.concatenate(
            [attn[0:1, :], attn[1:, :] * m, jnp.zeros((2, nn), jnp.float32)],
            axis=0)
        cols.append(wts)
    out_ref[...] = jnp.concatenate(cols, axis=1)


@functools.partial(
    pl.kernel,
    mesh=plsc.VectorSubcoreMesh(core_axis_name="c", subcore_axis_name="s"),
    out_type=jax.ShapeDtypeStruct((4096, 128), jnp.float32),
    scratch_types=[
        pltpu.VMEM((6 * CHUNK,), jnp.int32),
        pltpu.VMEM((6 * CHUNK + 16,), jnp.float32),
        pltpu.VMEM((6 * CHUNK, 128), jnp.float32),
        pltpu.VMEM((CHUNK, 128), jnp.float32),
        pltpu.SemaphoreType.DMA,
    ],
)
def _sc_mix(feat_hbm, gidx_hbm, w_hbm, out_hbm, idx_v, w_v, rows_v, acc_v,
            sem):
    wid = lax.axis_index("s") * 2 + lax.axis_index("c")
    node0 = wid * (4096 // NTILES)

    def chunk_body(ci, carry):
        nbase = node0 + ci * CHUNK
        fbase = nbase * 6
        pltpu.sync_copy(gidx_hbm.at[pl.ds(fbase, 6 * CHUNK)], idx_v)
        pltpu.sync_copy(w_hbm.at[pl.ds(fbase, 6 * CHUNK)],
                        w_v.at[pl.ds(0, 6 * CHUNK)])
        pltpu.async_copy(feat_hbm.at[idx_v], rows_v, sem).wait()

        def node_body(nl, c2):
            row0 = nl * 6
            wv = w_v[pl.ds(row0, 16)]
            for l in range(8):
                sl = pl.ds(l * 16, 16)
                acc = wv[0] * rows_v[row0, sl]
                for c in range(1, 6):
                    acc = acc + wv[c] * rows_v[row0 + c, sl]
                acc_v[nl, sl] = acc
            return c2

        lax.fori_loop(0, CHUNK, node_body, 0)
        pltpu.sync_copy(acc_v, out_hbm.at[pl.ds(nbase, CHUNK)])
        return carry

    lax.fori_loop(0, (4096 // NTILES) // CHUNK, chunk_body, 0)


def _mm_kernel(mixed_ref, w_ref, out_ref):
    h = jnp.dot(mixed_ref[...], w_ref[...],
                preferred_element_type=jnp.float32)
    out_ref[...] = jnp.where(h >= 0, h, jnp.exp(jnp.minimum(h, 0.0)) - 1.0)


@jax.jit
def _run(feature, cxt_t, offsets, mask_t, W, a2d, gidx_flat):
    bs, nper = cxt_t.shape[0], cxt_t.shape[2]
    rows, out_f = bs * nper, W.shape[1]

    attn_t = pl.pallas_call(
        _attn_kernel,
        grid_spec=pltpu.PrefetchScalarGridSpec(
            num_scalar_prefetch=1,
            grid=(bs // BPB,),
            in_specs=[
                pl.BlockSpec((BPB, 5, nper), lambda g, *_: (g, 0, 0)),
                pl.BlockSpec((BPB, 5, nper), lambda g, *_: (g, 0, 0)),
                pl.BlockSpec(feature.shape, lambda g, *_: (0, 0)),
                pl.BlockSpec(W.shape, lambda g, *_: (0, 0)),
                pl.BlockSpec(a2d.shape, lambda g, *_: (0, 0)),
            ],
            out_specs=pl.BlockSpec((8, BPB * nper), lambda g, *_: (0, g)),
            scratch_shapes=[pltpu.VMEM((feature.shape[1], 2), jnp.float32)],
        ),
        out_shape=jax.ShapeDtypeStruct((8, rows), jnp.float32),
    )(offsets, cxt_t, mask_t, feature, W, a2d)

    w_flat = attn_t[:6].T.reshape(-1)
    mixed = _sc_mix(feature, gidx_flat, w_flat)

    return pl.pallas_call(
        _mm_kernel,
        grid=(bs // BPB,),
        in_specs=[
            pl.BlockSpec((BPB * nper, feature.shape[1]), lambda g: (g, 0)),
            pl.BlockSpec(W.shape, lambda g: (0, 0)),
        ],
        out_specs=pl.BlockSpec((BPB * nper, out_f), lambda g: (g, 0)),
        out_shape=jax.ShapeDtypeStruct((rows, out_f), jnp.float32),
    )(mixed, W)


def kernel(feature, cxt_idx, offset_idx, cxt_idx_mask, bs, n, W, a):
    out_f = W.shape[1]
    maskf = (cxt_idx_mask
             & (jnp.asarray(bs) > 0)
             & (jnp.asarray(n) > 0)).astype(jnp.float32)
    offsets = offset_idx.reshape(-1).astype(jnp.int32)
    a2d = a.reshape(2, out_f)
    rows = feature.shape[0]
    self_idx = jnp.arange(rows, dtype=jnp.int32)[:, None]
    nbr_idx = (offset_idx + cxt_idx).reshape(rows, 5).astype(jnp.int32)
    gidx_flat = jnp.concatenate([self_idx, nbr_idx], axis=1).reshape(-1)
    return _run(feature, cxt_idx.transpose(0, 2, 1), offsets,
                maskf.transpose(0, 2, 1), W, a2d, gidx_flat)


# speedup vs baseline: 2.9258x; 2.9258x over previous
import functools

import jax
import jax.numpy as jnp
from jax.experimental import pallas as pl
from jax.experimental.pallas import tpu as pltpu

ALPHA = 0.2
BPB = 16
WIN = 72


def _gat_kernel(offsets_ref, cxt_ref, mask_ref, feat_ref, w_ref, a2_ref,
                out_ref, wa_ref):
    g = pl.program_id(0)
    nn = cxt_ref.shape[2]
    nrows = feat_ref.shape[0]

    @pl.when(g == 0)
    def _():
        wa_ref[...] = jnp.dot(w_ref[:], a2_ref[:].T,
                              preferred_element_type=jnp.float32)

    wa = wa_ref[...]

    feat_prog = feat_ref[pl.ds(g * BPB * nn, BPB * nn), :]
    st_t = jax.lax.dot_general(wa, feat_prog, (((0,), (1,)), ((), ())),
                               preferred_element_type=jnp.float32)

    iota_sub = jax.lax.broadcasted_iota(jnp.int32, (1, WIN, 1), 1)
    eye = (jax.lax.broadcasted_iota(jnp.int32, (nn, nn), 0)
           == jax.lax.broadcasted_iota(jnp.int32, (nn, nn), 1))
    mixed_parts = []
    for k in range(BPB):
        b = g * BPB + k
        off = offsets_ref[b]
        base = jnp.minimum((off // 8) * 8, nrows - WIN)
        r = off - base

        feat_b = feat_prog[k * nn:(k + 1) * nn, :]
        win = feat_ref[pl.ds(base, WIN), :]
        tw = jnp.dot(win, wa[:, 1:2], preferred_element_type=jnp.float32)

        cxt = cxt_ref[k] + r
        m = mask_ref[k]

        ohm = jnp.where(cxt[:, None, :] == iota_sub, m[:, None, :], 0.0)
        tg = jnp.sum(ohm * tw[None, :, :], axis=1)

        s_row = st_t[0:1, k * nn:(k + 1) * nn]
        t_row = st_t[1:2, k * nn:(k + 1) * nn]
        e = jnp.concatenate([s_row + t_row, s_row + tg], axis=0)
        e = jnp.where(e >= 0, e, ALPHA * e)
        e = e - jnp.max(e, axis=0, keepdims=True)
        ex = jnp.exp(e)
        attn = ex / jnp.sum(ex, axis=0, keepdims=True)

        s_mat = ohm[0] * attn[1:2, :]
        for j in range(1, 5):
            s_mat = s_mat + ohm[j] * attn[j + 1:j + 2, :]
        s_self = jnp.where(eye, attn[0:1, :], 0.0)
        mixed_parts.append(
            jax.lax.dot_general(s_mat, win, (((0,), (0,)), ((), ())),
                                preferred_element_type=jnp.float32)
            + jax.lax.dot_general(s_self, feat_b, (((0,), (0,)), ((), ())),
                                  preferred_element_type=jnp.float32))

    mixed = jnp.concatenate(mixed_parts, axis=0)
    h = jnp.dot(mixed, w_ref[:], preferred_element_type=jnp.float32)
    out_ref[...] = jnp.where(h >= 0, h, jnp.exp(jnp.minimum(h, 0.0)) - 1.0)


@jax.jit
def _run(feature, cxt_t, offsets, mask_t, W, a2d):
    bs, nper = cxt_t.shape[0], cxt_t.shape[2]
    out_f = W.shape[1]
    grid_spec = pltpu.PrefetchScalarGridSpec(
        num_scalar_prefetch=1,
        grid=(bs // BPB,),
        in_specs=[
            pl.BlockSpec((BPB, 5, nper), lambda g, *_: (g, 0, 0)),
            pl.BlockSpec((BPB, 5, nper), lambda g, *_: (g, 0, 0)),
            pl.BlockSpec(feature.shape, lambda g, *_: (0, 0)),
            pl.BlockSpec(W.shape, lambda g, *_: (0, 0)),
            pl.BlockSpec(a2d.shape, lambda g, *_: (0, 0)),
        ],
        out_specs=pl.BlockSpec((BPB * nper, out_f), lambda g, *_: (g, 0)),
        scratch_shapes=[pltpu.VMEM((feature.shape[1], 2), jnp.float32)],
    )
    return pl.pallas_call(
        _gat_kernel,
        grid_spec=grid_spec,
        out_shape=jax.ShapeDtypeStruct((bs * nper, out_f), jnp.float32),
    )(offsets, cxt_t, mask_t, feature, W, a2d)


def kernel(feature, cxt_idx, offset_idx, cxt_idx_mask, bs, n, W, a):
    out_f = W.shape[1]
    maskf = (cxt_idx_mask
             & (jnp.asarray(bs) > 0)
             & (jnp.asarray(n) > 0)).astype(jnp.float32)
    offsets = offset_idx.reshape(-1).astype(jnp.int32)
    a2d = a.reshape(2, out_f)
    return _run(feature, cxt_idx.transpose(0, 2, 1), offsets,
                maskf.transpose(0, 2, 1), W, a2d)
